# baseline (speedup 1.0000x reference)
import functools

import jax
import jax.numpy as jnp
from jax.experimental import pallas as pl
from jax.experimental.pallas import tpu as pltpu


def _row_tile(n: int, target: int) -> int:
    best = n
    for t in range(8, target + 1, 8):
        if n % t == 0:
            best = t
    return best


def _fused_kernel(nblocks, a_ref, adj_ref, k_ref, q_ref, x_ref, W1_ref,
                  W2_ref, linW_ref, linb_ref, Wg_ref, out_ref,
                  s1_ref, s2_ref, g_ref):
    i = pl.program_id(0)
    r = adj_ref.shape[0]

    @pl.when(i == 0)
    def _():
        s1 = jnp.dot(x_ref[...], W1_ref[...].astype(jnp.bfloat16),
                     preferred_element_type=jnp.float32)
        s1_ref[...] = s1.astype(jnp.bfloat16)
        logits = jnp.dot(x_ref[...], linW_ref[...].astype(jnp.bfloat16),
                         preferred_element_type=jnp.float32) + linb_ref[...]
        wave = jax.nn.softmax(logits, axis=-1)
        g = jnp.dot(wave.astype(jnp.bfloat16), Wg_ref[...].astype(jnp.bfloat16),
                    preferred_element_type=jnp.float32)
        g_ref[...] = g.astype(jnp.bfloat16)

    adjb = adj_ref[...].astype(jnp.bfloat16)

    @pl.when(i < nblocks)
    def _():
        h = jnp.dot(adjb, s1_ref[...], preferred_element_type=jnp.float32)
        h = jnp.maximum(h, 0.0)
        s2 = jnp.dot(h.astype(jnp.bfloat16), W2_ref[...].astype(jnp.bfloat16),
                     preferred_element_type=jnp.float32)
        blk = jnp.minimum(i, nblocks - 1)
        s2_ref[pl.ds(blk * r, r), :] = s2.astype(jnp.bfloat16)

    @pl.when(i >= nblocks)
    def _():
        a = a_ref[0]
        acc1 = jnp.dot(adjb, s2_ref[...], preferred_element_type=jnp.float32)
        qv = q_ref[...]
        m = (qv + a * (k_ref[...] - qv)).astype(jnp.bfloat16)
        acc2 = jnp.dot(m, g_ref[...], preferred_element_type=jnp.float32)
        out_ref[...] = jax.nn.softmax(acc1, axis=-1) + acc2


@functools.partial(jax.jit, static_argnames=())
def kernel(x, adj, q, k, W1, W2, lin_W, lin_b, Wg, apha):
    n, nfeat = x.shape
    nhid = W1.shape[1]
    nclass = W2.shape[1]

    a_sig = jax.nn.sigmoid(apha).reshape((1,))
    lin_b2 = lin_b.reshape((1, nclass))
    x_bf = x.astype(jnp.bfloat16)

    r = _row_tile(n, 200)
    p = n // r

    def adj_map(i):
        return (jnp.where(i < p, i, i - p), 0)

    def kq_map(i):
        return (jnp.maximum(i, p) - p, 0)

    def out_map(i):
        return (jnp.maximum(i, p) - p, 0)

    def const_map(i):
        return (0, 0)

    emb = pl.pallas_call(
        functools.partial(_fused_kernel, p),
        grid=(2 * p,),
        in_specs=[
            pl.BlockSpec(memory_space=pltpu.SMEM),
            pl.BlockSpec((r, n), adj_map),
            pl.BlockSpec((r, n), kq_map),
            pl.BlockSpec((r, n), kq_map),
            pl.BlockSpec((n, nfeat), const_map),
            pl.BlockSpec((nfeat, nhid), const_map),
            pl.BlockSpec((nhid, nclass), const_map),
            pl.BlockSpec((nfeat, nclass), const_map),
            pl.BlockSpec((1, nclass), const_map),
            pl.BlockSpec((nclass, nclass), const_map),
        ],
        out_specs=pl.BlockSpec((r, nclass), out_map),
        out_shape=jax.ShapeDtypeStruct((n, nclass), jnp.float32),
        scratch_shapes=[
            pltpu.VMEM((n, nhid), jnp.bfloat16),
            pltpu.VMEM((n, nclass), jnp.bfloat16),
            pltpu.VMEM((n, nclass), jnp.bfloat16),
        ],
        compiler_params=pltpu.CompilerParams(
            vmem_limit_bytes=62 * 1024 * 1024),
    )(a_sig, adj, k, q, x_bf, W1, W2, lin_W, lin_b2, Wg)
    return emb


# speedup vs baseline: 1.0159x; 1.0159x over previous
import functools

import jax
import jax.numpy as jnp
from jax.experimental import pallas as pl
from jax.experimental.pallas import tpu as pltpu


def _row_tile(n: int, target: int) -> int:
    best = n
    for t in range(8, target + 1, 8):
        if n % t == 0:
            best = t
    return best


def _gc_kernel(adj_ref, x_ref, W1_ref, linW_ref, linb_ref, Wg_ref,
               s2_ref, g_ref, s1_ref):
    @pl.when(pl.program_id(0) == 0)
    def _():
        s1 = jnp.dot(x_ref[...], W1_ref[...].astype(jnp.bfloat16),
                     preferred_element_type=jnp.float32)
        s1_ref[...] = s1.astype(jnp.bfloat16)
        logits = jnp.dot(x_ref[...], linW_ref[...].astype(jnp.bfloat16),
                         preferred_element_type=jnp.float32) + linb_ref[...]
        wave = jax.nn.softmax(logits, axis=-1)
        g = jnp.dot(wave.astype(jnp.bfloat16), Wg_ref[...].astype(jnp.bfloat16),
                    preferred_element_type=jnp.float32)
        g_ref[...] = g.astype(jnp.bfloat16)

    adjb = adj_ref[...].astype(jnp.bfloat16)
    h = jnp.dot(adjb, s1_ref[...], preferred_element_type=jnp.float32)
    h = jnp.maximum(h, 0.0)
    s2_ref[...] = h.astype(jnp.bfloat16)


def _emb_kernel(a_ref, x1_ref, W2_ref, k_ref, q_ref, adj_ref, g_ref,
                out_ref, s2_ref):
    @pl.when(pl.program_id(0) == 0)
    def _():
        s2 = jnp.dot(x1_ref[...], W2_ref[...].astype(jnp.bfloat16),
                     preferred_element_type=jnp.float32)
        s2_ref[...] = s2.astype(jnp.bfloat16)

    a = a_ref[0]
    adjb = adj_ref[...].astype(jnp.bfloat16)
    acc1 = jnp.dot(adjb, s2_ref[...], preferred_element_type=jnp.float32)
    qv = q_ref[...]
    m = (qv + a * (k_ref[...] - qv)).astype(jnp.bfloat16)
    acc2 = jnp.dot(m, g_ref[...], preferred_element_type=jnp.float32)
    out_ref[...] = jax.nn.softmax(acc1, axis=-1) + acc2


@functools.partial(jax.jit, static_argnames=())
def kernel(x, adj, q, k, W1, W2, lin_W, lin_b, Wg, apha):
    n, nfeat = x.shape
    nhid = W1.shape[1]
    nclass = W2.shape[1]

    a_sig = jax.nn.sigmoid(apha).reshape((1,))
    lin_b2 = lin_b.reshape((1, nclass))
    x_bf = x.astype(jnp.bfloat16)

    r1 = _row_tile(n, 400)
    x1, g = pl.pallas_call(
        _gc_kernel,
        grid=(n // r1,),
        in_specs=[
            pl.BlockSpec((r1, n), lambda i: (i, 0)),
            pl.BlockSpec((n, nfeat), lambda i: (0, 0)),
            pl.BlockSpec((nfeat, nhid), lambda i: (0, 0)),
            pl.BlockSpec((nfeat, nclass), lambda i: (0, 0)),
            pl.BlockSpec((1, nclass), lambda i: (0, 0)),
            pl.BlockSpec((nclass, nclass), lambda i: (0, 0)),
        ],
        out_specs=[
            pl.BlockSpec((r1, nhid), lambda i: (i, 0)),
            pl.BlockSpec((n, nclass), lambda i: (0, 0)),
        ],
        out_shape=[
            jax.ShapeDtypeStruct((n, nhid), jnp.bfloat16),
            jax.ShapeDtypeStruct((n, nclass), jnp.bfloat16),
        ],
        scratch_shapes=[pltpu.VMEM((n, nhid), jnp.bfloat16)],
        compiler_params=pltpu.CompilerParams(
            vmem_limit_bytes=62 * 1024 * 1024),
    )(adj, x_bf, W1, lin_W, lin_b2, Wg)

    r2 = _row_tile(n, 200)
    emb = pl.pallas_call(
        _emb_kernel,
        grid=(n // r2,),
        in_specs=[
            pl.BlockSpec(memory_space=pltpu.SMEM),
            pl.BlockSpec((n, nhid), lambda i: (0, 0)),
            pl.BlockSpec((nhid, nclass), lambda i: (0, 0)),
            pl.BlockSpec((r2, n), lambda i: (i, 0)),
            pl.BlockSpec((r2, n), lambda i: (i, 0)),
            pl.BlockSpec((r2, n), lambda i: (i, 0)),
            pl.BlockSpec((n, nclass), lambda i: (0, 0)),
        ],
        out_specs=pl.BlockSpec((r2, nclass), lambda i: (i, 0)),
        out_shape=jax.ShapeDtypeStruct((n, nclass), jnp.float32),
        scratch_shapes=[pltpu.VMEM((n, nclass), jnp.bfloat16)],
        compiler_params=pltpu.CompilerParams(
            vmem_limit_bytes=62 * 1024 * 1024),
    )(a_sig, x1, W2, k, q, adj, g)
    return emb


# speedup vs baseline: 1.0270x; 1.0109x over previous
import functools

import jax
import jax.numpy as jnp
from jax.experimental import pallas as pl
from jax.experimental.pallas import tpu as pltpu


def _row_tile(n: int, target: int) -> int:
    best = n
    for t in range(8, target + 1, 8):
        if n % t == 0:
            best = t
    return best


def _gc_kernel(adj_ref, x_ref, W1_ref, W2_ref, linW_ref, linb_ref, Wg_ref,
               s2_ref, g_ref, s1_ref):
    @pl.when(pl.program_id(0) == 0)
    def _():
        xb = x_ref[...].astype(jnp.bfloat16)
        s1 = jnp.dot(xb, W1_ref[...].astype(jnp.bfloat16),
                     preferred_element_type=jnp.float32)
        s1_ref[...] = s1.astype(jnp.bfloat16)
        logits = jnp.dot(xb, linW_ref[...].astype(jnp.bfloat16),
                         preferred_element_type=jnp.float32) + linb_ref[...]
        wave = jax.nn.softmax(logits, axis=-1)
        g = jnp.dot(wave.astype(jnp.bfloat16), Wg_ref[...].astype(jnp.bfloat16),
                    preferred_element_type=jnp.float32)
        g_ref[...] = g.astype(jnp.bfloat16)

    adjb = adj_ref[...].astype(jnp.bfloat16)
    h = jnp.dot(adjb, s1_ref[...], preferred_element_type=jnp.float32)
    h = jnp.maximum(h, 0.0)
    s2 = jnp.dot(h.astype(jnp.bfloat16), W2_ref[...].astype(jnp.bfloat16),
                 preferred_element_type=jnp.float32)
    s2_ref[...] = s2.astype(jnp.bfloat16)


def _emb_kernel(a_ref, adj_ref, k_ref, q_ref, s2_ref, g_ref, out_ref):
    a = a_ref[0]
    adjb = adj_ref[...].astype(jnp.bfloat16)
    acc1 = jnp.dot(adjb, s2_ref[...], preferred_element_type=jnp.float32)
    qv = q_ref[...]
    m = (qv + a * (k_ref[...] - qv)).astype(jnp.bfloat16)
    acc2 = jnp.dot(m, g_ref[...], preferred_element_type=jnp.float32)
    out_ref[...] = jax.nn.softmax(acc1, axis=-1) + acc2


@functools.partial(jax.jit, static_argnames=())
def kernel(x, adj, q, k, W1, W2, lin_W, lin_b, Wg, apha):
    n, nfeat = x.shape
    nhid = W1.shape[1]
    nclass = W2.shape[1]

    a_sig = jax.nn.sigmoid(apha).reshape((1,))
    lin_b2 = lin_b.reshape((1, nclass))

    r1 = _row_tile(n, 400)
    s2, g = pl.pallas_call(
        _gc_kernel,
        grid=(n // r1,),
        in_specs=[
            pl.BlockSpec((r1, n), lambda i: (i, 0)),
            pl.BlockSpec((n, nfeat), lambda i: (0, 0)),
            pl.BlockSpec((nfeat, nhid), lambda i: (0, 0)),
            pl.BlockSpec((nhid, nclass), lambda i: (0, 0)),
            pl.BlockSpec((nfeat, nclass), lambda i: (0, 0)),
            pl.BlockSpec((1, nclass), lambda i: (0, 0)),
            pl.BlockSpec((nclass, nclass), lambda i: (0, 0)),
        ],
        out_specs=[
            pl.BlockSpec((r1, nclass), lambda i: (i, 0)),
            pl.BlockSpec((n, nclass), lambda i: (0, 0)),
        ],
        out_shape=[
            jax.ShapeDtypeStruct((n, nclass), jnp.bfloat16),
            jax.ShapeDtypeStruct((n, nclass), jnp.bfloat16),
        ],
        scratch_shapes=[pltpu.VMEM((n, nhid), jnp.bfloat16)],
        compiler_params=pltpu.CompilerParams(
            vmem_limit_bytes=62 * 1024 * 1024),
    )(adj, x, W1, W2, lin_W, lin_b2, Wg)

    r2 = _row_tile(n, 200)
    emb = pl.pallas_call(
        _emb_kernel,
        grid=(n // r2,),
        in_specs=[
            pl.BlockSpec(memory_space=pltpu.SMEM),
            pl.BlockSpec((r2, n), lambda i: (i, 0)),
            pl.BlockSpec((r2, n), lambda i: (i, 0)),
            pl.BlockSpec((r2, n), lambda i: (i, 0)),
            pl.BlockSpec((n, nclass), lambda i: (0, 0)),
            pl.BlockSpec((n, nclass), lambda i: (0, 0)),
        ],
        out_specs=pl.BlockSpec((r2, nclass), lambda i: (i, 0)),
        out_shape=jax.ShapeDtypeStruct((n, nclass), jnp.float32),
        compiler_params=pltpu.CompilerParams(
            vmem_limit_bytes=62 * 1024 * 1024),
    )(a_sig, adj, k, q, s2, g)
    return emb


# speedup vs baseline: 1.0304x; 1.0034x over previous
import functools

import jax
import jax.numpy as jnp
from jax.experimental import pallas as pl
from jax.experimental.pallas import tpu as pltpu


def _row_tile(n: int, target: int) -> int:
    best = n
    for t in range(8, target + 1, 8):
        if n % t == 0:
            best = t
    return best


def _gc_kernel(adj_ref, x_ref, W1_ref, W2_ref, linW_ref, linb_ref, Wg_ref,
               s2_ref, g_ref, s1_ref):
    @pl.when(pl.program_id(0) == 0)
    def _():
        xb = x_ref[...].astype(jnp.bfloat16)
        s1 = jnp.dot(xb, W1_ref[...].astype(jnp.bfloat16),
                     preferred_element_type=jnp.float32)
        s1_ref[...] = s1.astype(jnp.bfloat16)
        logits = jnp.dot(xb, linW_ref[...].astype(jnp.bfloat16),
                         preferred_element_type=jnp.float32) + linb_ref[...]
        wave = jax.nn.softmax(logits, axis=-1)
        g = jnp.dot(wave.astype(jnp.bfloat16), Wg_ref[...].astype(jnp.bfloat16),
                    preferred_element_type=jnp.float32)
        g_ref[...] = g.astype(jnp.bfloat16)

    adjb = adj_ref[...].astype(jnp.bfloat16)
    h = jnp.dot(adjb, s1_ref[...], preferred_element_type=jnp.float32)
    h = jnp.maximum(h, 0.0)
    s2 = jnp.dot(h.astype(jnp.bfloat16), W2_ref[...].astype(jnp.bfloat16),
                 preferred_element_type=jnp.float32)
    s2_ref[...] = s2.astype(jnp.bfloat16)


def _emb_kernel(a_ref, adj_ref, k_ref, q_ref, s2_ref, g_ref, out_ref):
    a = jax.nn.sigmoid(a_ref[...])[0, 0]
    adjb = adj_ref[...].astype(jnp.bfloat16)
    acc1 = jnp.dot(adjb, s2_ref[...], preferred_element_type=jnp.float32)
    qv = q_ref[...]
    m = (qv + a * (k_ref[...] - qv)).astype(jnp.bfloat16)
    acc2 = jnp.dot(m, g_ref[...], preferred_element_type=jnp.float32)
    out_ref[...] = jax.nn.softmax(acc1, axis=-1) + acc2


@functools.partial(jax.jit, static_argnames=())
def kernel(x, adj, q, k, W1, W2, lin_W, lin_b, Wg, apha):
    n, nfeat = x.shape
    nhid = W1.shape[1]
    nclass = W2.shape[1]

    a_sig = apha.reshape((1, 1))
    lin_b2 = lin_b.reshape((1, nclass))

    r1 = _row_tile(n, 400)
    s2, g = pl.pallas_call(
        _gc_kernel,
        grid=(n // r1,),
        in_specs=[
            pl.BlockSpec((r1, n), lambda i: (i, 0)),
            pl.BlockSpec((n, nfeat), lambda i: (0, 0)),
            pl.BlockSpec((nfeat, nhid), lambda i: (0, 0)),
            pl.BlockSpec((nhid, nclass), lambda i: (0, 0)),
            pl.BlockSpec((nfeat, nclass), lambda i: (0, 0)),
            pl.BlockSpec((1, nclass), lambda i: (0, 0)),
            pl.BlockSpec((nclass, nclass), lambda i: (0, 0)),
        ],
        out_specs=[
            pl.BlockSpec((r1, nclass), lambda i: (i, 0)),
            pl.BlockSpec((n, nclass), lambda i: (0, 0)),
        ],
        out_shape=[
            jax.ShapeDtypeStruct((n, nclass), jnp.bfloat16),
            jax.ShapeDtypeStruct((n, nclass), jnp.bfloat16),
        ],
        scratch_shapes=[pltpu.VMEM((n, nhid), jnp.bfloat16)],
        compiler_params=pltpu.CompilerParams(
            vmem_limit_bytes=62 * 1024 * 1024),
    )(adj, x, W1, W2, lin_W, lin_b2, Wg)

    r2 = _row_tile(n, 200)
    emb = pl.pallas_call(
        _emb_kernel,
        grid=(n // r2,),
        in_specs=[
            pl.BlockSpec((1, 1), lambda i: (0, 0)),
            pl.BlockSpec((r2, n), lambda i: (i, 0)),
            pl.BlockSpec((r2, n), lambda i: (i, 0)),
            pl.BlockSpec((r2, n), lambda i: (i, 0)),
            pl.BlockSpec((n, nclass), lambda i: (0, 0)),
            pl.BlockSpec((n, nclass), lambda i: (0, 0)),
        ],
        out_specs=pl.BlockSpec((r2, nclass), lambda i: (i, 0)),
        out_shape=jax.ShapeDtypeStruct((n, nclass), jnp.float32),
        compiler_params=pltpu.CompilerParams(
            vmem_limit_bytes=62 * 1024 * 1024),
    )(a_sig, adj, k, q, s2, g)
    return emb
